# baseline (speedup 1.0000x reference)
import functools

import jax
import jax.numpy as jnp
from jax.experimental import pallas as pl
from jax.experimental.pallas import tpu as pltpu


def _add_body(tid_ref, mesh_ref, emb_ref, out_ref):
    out_ref[...] = mesh_ref[0] + emb_ref[0]


@functools.partial(jax.jit, static_argnums=())
def _run(time_vector, mesh_flat, embedding_weight):
    n_times, _, flat = mesh_flat.shape
    time_id = jnp.round(time_vector[0, 0] * (n_times - 1)).astype(jnp.int32)
    tid = time_id[None]

    block_w = 32768
    grid = (flat + block_w - 1) // block_w

    grid_spec = pltpu.PrefetchScalarGridSpec(
        num_scalar_prefetch=1,
        grid=(grid,),
        in_specs=[
            pl.BlockSpec((1, 1, block_w), lambda i, tid_ref: (tid_ref[0], 0, i)),
            pl.BlockSpec((1, 1, block_w), lambda i, tid_ref: (tid_ref[0], 0, i)),
        ],
        out_specs=pl.BlockSpec((1, block_w), lambda i, tid_ref: (0, i)),
    )
    out = pl.pallas_call(
        _add_body,
        grid_spec=grid_spec,
        out_shape=jax.ShapeDtypeStruct((1, flat), jnp.float32),
    )(tid, mesh_flat, embedding_weight)
    return out


def kernel(time_vector, mesh_predictions, embedding_weight):
    n_times, n_nodes, _ = mesh_predictions.shape
    mesh_flat = mesh_predictions.reshape(n_times, 1, n_nodes * 3)
    emb = embedding_weight.reshape(n_times, 1, n_nodes * 3)
    out = _run(time_vector, mesh_flat, emb)
    return out.reshape(n_nodes, 3)


# speedup vs baseline: 45.2515x; 45.2515x over previous
import functools

import jax
import jax.numpy as jnp
from jax import lax
from jax.experimental import pallas as pl
from jax.experimental.pallas import tpu as pltpu
from jax.experimental.pallas import tpu_sc as plsc

_NW = 32
_CHUNK = 3200
_TAIL = 768
_NODES_SC = _CHUNK * (_NW - 1) + _TAIL


def _sc_body(tid_hbm, mesh_hbm, emb_hbm, out_hbm,
             idxv, ebuf, mb0, mb1, mb2, obuf, se, s0, s1, s2):
    w = lax.axis_index("s") * 2 + lax.axis_index("c")
    pltpu.sync_copy(tid_hbm, idxv)
    base = w * _CHUNK
    mbufs = (mb0, mb1, mb2)
    sems = (s0, s1, s2)

    @pl.when(w < _NW - 1)
    def _in_full():
        ce = pltpu.async_copy(
            emb_hbm.at[idxv, pl.ds(3 * base, 3 * _CHUNK)], ebuf, se
        )
        cs = [
            pltpu.async_copy(
                mesh_hbm.at[c].at[idxv, pl.ds(base, _CHUNK)], mbufs[c], sems[c]
            )
            for c in range(3)
        ]
        ce.wait()
        for h in cs:
            h.wait()

    @pl.when(w == _NW - 1)
    def _in_tail():
        ce = pltpu.async_copy(
            emb_hbm.at[idxv, pl.ds(3 * base, 3 * _TAIL)],
            ebuf.at[:, pl.ds(0, 3 * _TAIL)],
            se,
        )
        cs = [
            pltpu.async_copy(
                mesh_hbm.at[c].at[idxv, pl.ds(base, _TAIL)],
                mbufs[c].at[:, pl.ds(0, _TAIL)],
                sems[c],
            )
            for c in range(3)
        ]
        ce.wait()
        for h in cs:
            h.wait()

    lanes = lax.iota(jnp.int32, 16)
    zeros = lanes * 0
    n_vec = jnp.where(w == _NW - 1, _TAIL // 16, _CHUNK // 16)

    def body(v, carry):
        off = v * 16
        for c in range(3):
            idx = lanes * 3 + (off * 3 + c)
            ev = plsc.load_gather(ebuf, [zeros, idx])
            mv = mbufs[c][0, pl.ds(off, 16)]
            obuf[c, pl.ds(off, 16)] = ev + mv
        return carry

    lax.fori_loop(0, n_vec, body, 0)

    @pl.when(w < _NW - 1)
    def _out_full():
        pltpu.sync_copy(obuf, out_hbm.at[:, pl.ds(base, _CHUNK)])

    @pl.when(w == _NW - 1)
    def _out_tail():
        pltpu.sync_copy(
            obuf.at[:, pl.ds(0, _TAIL)],
            out_hbm.at[:, pl.ds(base, _TAIL)],
        )


@jax.jit
def _run(time_vector, mesh_t, embedding_weight):
    _, n_times, n_nodes = mesh_t.shape
    time_id = jnp.round(time_vector[0, 0] * (n_times - 1)).astype(jnp.int32)
    tid1 = time_id[None]

    kern = pl.kernel(
        _sc_body,
        mesh=plsc.VectorSubcoreMesh(core_axis_name="c", subcore_axis_name="s"),
        out_type=jax.ShapeDtypeStruct((3, n_nodes), jnp.float32),
        compiler_params=pltpu.CompilerParams(needs_layout_passes=False),
        scratch_types=[
            pltpu.VMEM((1,), jnp.int32),
            pltpu.VMEM((1, 3 * _CHUNK), jnp.float32),
            pltpu.VMEM((1, _CHUNK), jnp.float32),
            pltpu.VMEM((1, _CHUNK), jnp.float32),
            pltpu.VMEM((1, _CHUNK), jnp.float32),
            pltpu.VMEM((3, _CHUNK), jnp.float32),
            pltpu.SemaphoreType.DMA,
            pltpu.SemaphoreType.DMA,
            pltpu.SemaphoreType.DMA,
            pltpu.SemaphoreType.DMA,
        ],
    )
    out = kern(tid1, mesh_t, embedding_weight)

    tm = lax.dynamic_slice(mesh_t, (0, time_id, _NODES_SC), (3, 1, 32))[:, 0, :]
    te = lax.dynamic_slice(embedding_weight, (time_id, 3 * _NODES_SC), (1, 96))
    tail = tm + te.reshape(32, 3).T
    return lax.dynamic_update_slice(out, tail, (0, _NODES_SC))


def kernel(time_vector, mesh_predictions, embedding_weight):
    mesh_t = jnp.transpose(mesh_predictions, (2, 0, 1))
    out = _run(time_vector, mesh_t, embedding_weight)
    return out.T
